# baseline (speedup 1.0000x reference)
import functools

import jax
import jax.numpy as jnp
from jax import lax
from jax.experimental import pallas as pl
from jax.experimental.pallas import tpu as pltpu
from jax.experimental.pallas import tpu_sc as plsc

_EPS = 1e-6
_MARGIN = 1.0


def _tc_transpose(xt):
    D, N = xt.shape
    C = 8192
    NB = (N + C - 1) // C

    def body(in_ref, out_ref):
        out_ref[...] = in_ref[...].T

    return pl.pallas_call(
        body,
        grid=(NB,),
        in_specs=[pl.BlockSpec((D, C), lambda n: (0, n))],
        out_specs=pl.BlockSpec((C, D), lambda n: (n, 0)),
        out_shape=jax.ShapeDtypeStruct((N, D), jnp.float32),
    )(xt)


def _sc_gather(U_true, U2, V, users, pos, negT, B, DIM, NNEG):
    info = plsc.get_sparse_core_info()
    NC, NS = info.num_cores, info.num_subcores
    NW = NC * NS
    bw = B // NW
    nn = (B * NNEG) // NW
    half = nn // 2
    mesh = plsc.VectorSubcoreMesh(core_axis_name="c", subcore_axis_name="s")

    @functools.partial(
        pl.kernel,
        mesh=mesh,
        out_type=[
            jax.ShapeDtypeStruct((B, DIM), jnp.float32),
            jax.ShapeDtypeStruct((B, 2 * DIM), jnp.float32),
            jax.ShapeDtypeStruct((B, DIM), jnp.float32),
            jax.ShapeDtypeStruct((B * NNEG, DIM), jnp.float32),
        ],
        scratch_types=[
            pltpu.VMEM((bw,), jnp.int32),
            pltpu.VMEM((half,), jnp.int32),
            pltpu.VMEM((bw, DIM), jnp.float32),
            pltpu.VMEM((bw, 2 * DIM), jnp.float32),
            pltpu.VMEM((half, DIM), jnp.float32),
            pltpu.SemaphoreType.DMA,
        ],
        compiler_params=pltpu.CompilerParams(use_tc_tiling_on_sc=False),
    )
    def k(ut_hbm, u2_hbm, v_hbm, users_hbm, pos_hbm, negT_hbm,
          ut_out, uu_out, i_out, j_out, idx_u, idx_n, row_v, uu_v, j_v, sem):
        wid = lax.axis_index("s") * NC + lax.axis_index("c")
        base = wid * bw
        pltpu.sync_copy(users_hbm.at[pl.ds(base, bw)], idx_u)
        c1 = pltpu.async_copy(ut_hbm.at[idx_u], row_v, sem)
        c2 = pltpu.async_copy(u2_hbm.at[idx_u], uu_v, sem)
        c1.wait()
        c2.wait()
        pltpu.sync_copy(row_v, ut_out.at[pl.ds(base, bw)])
        pltpu.sync_copy(uu_v, uu_out.at[pl.ds(base, bw)])
        pltpu.sync_copy(pos_hbm.at[pl.ds(base, bw)], idx_u)
        pltpu.async_copy(v_hbm.at[idx_u], row_v, sem).wait()
        pltpu.sync_copy(row_v, i_out.at[pl.ds(base, bw)])
        nbase = wid * nn
        for c in range(2):
            off = nbase + c * half
            pltpu.sync_copy(negT_hbm.at[pl.ds(off, half)], idx_n)
            pltpu.async_copy(v_hbm.at[idx_n], j_v, sem).wait()
            pltpu.sync_copy(j_v, j_out.at[pl.ds(off, half)])

    return k(U_true, U2, V, users, pos, negT)


def _tc_loss(ut, uu, i_rows, j3, das2, Wt, b2, da_pad, B, DIM, NNEG, DA):
    R = 2048
    NB = B // R
    NDA = da_pad.shape[0]

    def body(ut_ref, uu_ref, i_ref, j_ref, das_ref, w_ref, b_ref, dat_ref,
             out_ref):
        step = pl.program_id(0)
        s = 2.0 * ut_ref[...] + uu_ref[:, :DIM] + uu_ref[:, DIM:]
        das = jnp.minimum(jnp.maximum(das_ref[...], 0), DA)
        onehot = (das == lax.broadcasted_iota(jnp.int32, (R, NDA), 1)
                  ).astype(jnp.float32)
        u = (jnp.dot(s, w_ref[...], preferred_element_type=jnp.float32)
             + b_ref[...]
             + jnp.dot(onehot, dat_ref[...], preferred_element_type=jnp.float32))
        up = u + _EPS
        dpos = up - i_ref[...]
        dp = jnp.sqrt(jnp.sum(dpos * dpos, axis=1, keepdims=True))
        acc = jnp.zeros((), jnp.float32)
        for k in range(NNEG):
            dneg = up - j_ref[k]
            dn = jnp.sqrt(jnp.sum(dneg * dneg, axis=1, keepdims=True))
            acc = acc + jnp.sum(jnp.maximum(dp - dn + _MARGIN, 0.0))

        @pl.when(step == 0)
        def _():
            out_ref[...] = jnp.zeros_like(out_ref[...])

        out_ref[...] = out_ref[...] + acc * (1.0 / B)

    out = pl.pallas_call(
        body,
        grid=(NB,),
        in_specs=[
            pl.BlockSpec((R, DIM), lambda n: (n, 0)),
            pl.BlockSpec((R, 2 * DIM), lambda n: (n, 0)),
            pl.BlockSpec((R, DIM), lambda n: (n, 0)),
            pl.BlockSpec((NNEG, R, DIM), lambda n: (0, n, 0)),
            pl.BlockSpec((R, 1), lambda n: (n, 0)),
            pl.BlockSpec((DIM, DIM), lambda n: (0, 0)),
            pl.BlockSpec((1, DIM), lambda n: (0, 0)),
            pl.BlockSpec((NDA, DIM), lambda n: (0, 0)),
        ],
        out_specs=pl.BlockSpec((1, 1), lambda n: (0, 0)),
        out_shape=jax.ShapeDtypeStruct((1, 1), jnp.float32),
        compiler_params=pltpu.CompilerParams(
            dimension_semantics=("arbitrary",)),
    )(ut, uu, i_rows, j3, das2, Wt, b2, da_pad)
    return out[0, 0]


def kernel(phase, users, pos_job_ids, behavior_ids, das, neg_job_id_lists,
           U_true, U, V, da_tab, W, b):
    B = users.shape[0]
    DIM = U_true.shape[1]
    NNEG = neg_job_id_lists.shape[1]
    DA = da_tab.shape[0] - 1
    USER_SIZE = U.shape[0]

    user_cap = ((USER_SIZE + 127) // 128) * 128
    U_true_rm = _tc_transpose(U_true[:user_cap].T)
    U2_rm = _tc_transpose(U.reshape(USER_SIZE, 2 * DIM).T)
    V_rm = _tc_transpose(V.T)
    negT = neg_job_id_lists.T.reshape(-1)
    ut, uu, i_rows, j_rows = _sc_gather(
        U_true_rm, U2_rm, V_rm, users, pos_job_ids, negT, B, DIM, NNEG)
    j3 = j_rows.reshape(NNEG, B, DIM)
    das2 = das.reshape(B, 1)
    NDA = 128
    da_pad = jnp.pad(da_tab, ((0, NDA - (DA + 1)), (0, 0)))
    Wt = W.T
    b2 = b.reshape(1, DIM)
    return _tc_loss(ut, uu, i_rows, j3, das2, Wt, b2, da_pad,
                    B, DIM, NNEG, DA)


# speedup vs baseline: 2.7885x; 2.7885x over previous
import functools

import jax
import jax.numpy as jnp
from jax import lax
from jax.experimental import pallas as pl
from jax.experimental.pallas import tpu as pltpu
from jax.experimental.pallas import tpu_sc as plsc

_EPS = 1e-6
_MARGIN = 1.0


def _tc_transpose128(xt, chunk=8192, n_out=None):
    D, N = xt.shape
    if n_out is not None:
        N = n_out
    G = 128 // D
    C = min(chunk, ((N + 127) // 128) * 128)
    NB = (N + C - 1) // C
    Cf = C // G
    eye = jnp.eye(128, dtype=jnp.float32)

    def body(in_ref, eye_ref, out_ref):
        n = pl.program_id(0)

        def compute(mask_tail):
            acc = None
            for a in range(G):
                ta = lax.dot_general(
                    in_ref[:, a * Cf:(a + 1) * Cf],
                    eye_ref[a * D:(a + 1) * D, :],
                    (((0,), (0,)), ((), ())),
                    preferred_element_type=jnp.float32)
                if mask_tail:
                    lim = N - (NB - 1) * C - a * Cf
                    valid = (lax.broadcasted_iota(jnp.int32, (Cf, 128), 0)
                             < lim)
                    ta = jnp.where(valid, ta, 0.0)
                acc = ta if acc is None else acc + ta
            out_ref[...] = acc

        if N % C == 0:
            compute(False)
        else:
            @pl.when(n < NB - 1)
            def _():
                compute(False)

            @pl.when(n == NB - 1)
            def _():
                compute(True)

    return pl.pallas_call(
        body,
        grid=(NB,),
        in_specs=[
            pl.BlockSpec((D, C), lambda n: (0, n)),
            pl.BlockSpec((128, 128), lambda n: (0, 0)),
        ],
        out_specs=pl.BlockSpec((Cf, 128), lambda n: (n, 0)),
        out_shape=jax.ShapeDtypeStruct((NB * Cf, 128), jnp.float32),
    )(xt, eye)


def _sc_gather(T_ut, T_u2, T_v, T_da, users, pos, negT, das, B, NNEG, DA):
    info = plsc.get_sparse_core_info()
    NC, NS = info.num_cores, info.num_subcores
    NW = NC * NS
    bw = B // NW
    nn = (B * NNEG) // NW
    half = nn // 2
    mesh = plsc.VectorSubcoreMesh(core_axis_name="c", subcore_axis_name="s")

    @functools.partial(
        pl.kernel,
        mesh=mesh,
        out_type=[
            jax.ShapeDtypeStruct((B, 32), jnp.float32),
            jax.ShapeDtypeStruct((B, 32), jnp.float32),
            jax.ShapeDtypeStruct((B, 32), jnp.float32),
            jax.ShapeDtypeStruct((B, 32), jnp.float32),
            jax.ShapeDtypeStruct((B, 32), jnp.float32),
            jax.ShapeDtypeStruct((B * NNEG, 32), jnp.float32),
        ],
        scratch_types=[
            pltpu.VMEM((bw,), jnp.int32),
            pltpu.VMEM((bw,), jnp.int32),
            pltpu.VMEM((bw,), jnp.int32),
            pltpu.VMEM((bw,), jnp.int32),
            pltpu.VMEM((bw,), jnp.int32),
            pltpu.VMEM((half,), jnp.int32),
            pltpu.VMEM((5 * bw, 32), jnp.float32),
            pltpu.VMEM((half, 32), jnp.float32),
            pltpu.SemaphoreType.DMA,
        ],
        compiler_params=pltpu.CompilerParams(use_tc_tiling_on_sc=False),
    )
    def k(ut_hbm, u2_hbm, v_hbm, da_hbm, users_hbm, pos_hbm, negT_hbm,
          das_hbm, ut_out, g0_out, g1_out, i_out, da_out, j_out,
          idxA, idxB, idxC, idxP, idxD, idxN, rows_v, j_v, sem):
        wid = lax.axis_index("s") * NC + lax.axis_index("c")
        base = wid * bw
        pltpu.sync_copy(users_hbm.at[pl.ds(base, bw)], idxA)
        pltpu.sync_copy(pos_hbm.at[pl.ds(base, bw)], idxP)
        pltpu.sync_copy(das_hbm.at[pl.ds(base, bw)], idxD)

        def v32(r):
            return (jnp.right_shift(r, 13) * 8192
                    + (r & 2047) * 4 + (jnp.right_shift(r, 11) & 3))

        def xform(t, _):
            s = pl.ds(t * 16, 16)
            u = idxA[s]
            v0 = (jnp.right_shift(u, 13) * 16384 + (u & 4095) * 4
                  + (jnp.right_shift(u, 12) & 1) * 2)
            idxB[s] = v0
            idxC[s] = v0 + 1
            idxA[s] = v32(u)
            idxP[s] = v32(idxP[s])
            d = jnp.minimum(jnp.maximum(idxD[s], 0), DA)
            idxD[s] = (d & 31) * 4 + jnp.right_shift(d, 5)
            return _

        lax.fori_loop(0, bw // 16, xform, 0, unroll=4)

        c1 = pltpu.async_copy(ut_hbm.at[idxA], rows_v.at[pl.ds(0, bw)], sem)
        c2 = pltpu.async_copy(u2_hbm.at[idxB], rows_v.at[pl.ds(bw, bw)], sem)
        c3 = pltpu.async_copy(u2_hbm.at[idxC], rows_v.at[pl.ds(2 * bw, bw)],
                              sem)
        c4 = pltpu.async_copy(v_hbm.at[idxP], rows_v.at[pl.ds(3 * bw, bw)],
                              sem)
        c5 = pltpu.async_copy(da_hbm.at[idxD], rows_v.at[pl.ds(4 * bw, bw)],
                              sem)
        c1.wait()
        c2.wait()
        c3.wait()
        c4.wait()
        c5.wait()
        pltpu.sync_copy(rows_v.at[pl.ds(0, bw)], ut_out.at[pl.ds(base, bw)])
        pltpu.sync_copy(rows_v.at[pl.ds(bw, bw)], g0_out.at[pl.ds(base, bw)])
        pltpu.sync_copy(rows_v.at[pl.ds(2 * bw, bw)],
                        g1_out.at[pl.ds(base, bw)])
        pltpu.sync_copy(rows_v.at[pl.ds(3 * bw, bw)],
                        i_out.at[pl.ds(base, bw)])
        pltpu.sync_copy(rows_v.at[pl.ds(4 * bw, bw)],
                        da_out.at[pl.ds(base, bw)])

        nbase = wid * nn
        for c in range(2):
            off = nbase + c * half
            pltpu.sync_copy(negT_hbm.at[pl.ds(off, half)], idxN)

            def nxform(t, _):
                s = pl.ds(t * 16, 16)
                idxN[s] = v32(idxN[s])
                return _

            lax.fori_loop(0, half // 16, nxform, 0, unroll=4)
            pltpu.async_copy(v_hbm.at[idxN], j_v, sem).wait()
            pltpu.sync_copy(j_v, j_out.at[pl.ds(off, half)])

    return k(T_ut, T_u2, T_v, T_da, users, pos, negT, das)


def _tc_loss(ut_p, g0_p, g1_p, i_p, da_p, j_p, W128, b128, B, NNEG):
    Bp = B // 4
    Rp = 512
    NB = Bp // Rp

    def body(ut_ref, g0_ref, g1_ref, i_ref, da_ref, j_ref, w_ref, b_ref,
             out_ref):
        step = pl.program_id(0)
        s = 2.0 * ut_ref[...] + g0_ref[...] + g1_ref[...]
        u = (jnp.dot(s, w_ref[...], preferred_element_type=jnp.float32)
             + b_ref[...] + da_ref[...])
        up = u + _EPS
        gsel = (lax.broadcasted_iota(jnp.int32, (128, 4), 0) // 32
                == lax.broadcasted_iota(jnp.int32, (128, 4), 1)
                ).astype(jnp.float32)
        d0 = up - i_ref[...]
        dp = jnp.sqrt(jnp.dot(d0 * d0, gsel,
                              preferred_element_type=jnp.float32))
        acc = jnp.zeros((), jnp.float32)
        for k in range(NNEG):
            dk = up - j_ref[k]
            dn = jnp.sqrt(jnp.dot(dk * dk, gsel,
                                  preferred_element_type=jnp.float32))
            acc = acc + jnp.sum(jnp.maximum(dp - dn + _MARGIN, 0.0))

        @pl.when(step == 0)
        def _():
            out_ref[...] = jnp.zeros_like(out_ref[...])

        out_ref[...] = out_ref[...] + acc * (1.0 / B)

    out = pl.pallas_call(
        body,
        grid=(NB,),
        in_specs=[
            pl.BlockSpec((Rp, 128), lambda n: (n, 0)),
            pl.BlockSpec((Rp, 128), lambda n: (n, 0)),
            pl.BlockSpec((Rp, 128), lambda n: (n, 0)),
            pl.BlockSpec((Rp, 128), lambda n: (n, 0)),
            pl.BlockSpec((Rp, 128), lambda n: (n, 0)),
            pl.BlockSpec((NNEG, Rp, 128), lambda n: (0, n, 0)),
            pl.BlockSpec((128, 128), lambda n: (0, 0)),
            pl.BlockSpec((1, 128), lambda n: (0, 0)),
        ],
        out_specs=pl.BlockSpec((1, 1), lambda n: (0, 0)),
        out_shape=jax.ShapeDtypeStruct((1, 1), jnp.float32),
        compiler_params=pltpu.CompilerParams(
            dimension_semantics=("arbitrary",)),
    )(ut_p, g0_p, g1_p, i_p, da_p, j_p, W128, b128)
    return out[0, 0]


def kernel(phase, users, pos_job_ids, behavior_ids, das, neg_job_id_lists,
           U_true, U, V, da_tab, W, b):
    B = users.shape[0]
    DIM = U_true.shape[1]
    NNEG = neg_job_id_lists.shape[1]
    DA = da_tab.shape[0] - 1
    USER_SIZE = U.shape[0]
    JOB_SIZE = V.shape[0]

    user_cap = ((USER_SIZE + 127) // 128) * 128
    T_ut = _tc_transpose128(U_true.T, n_out=user_cap)
    T_u2 = _tc_transpose128(U.reshape(USER_SIZE, 2 * DIM).T)
    T_v = _tc_transpose128(V.T)
    da_padT = jnp.pad(da_tab, ((0, 128 - (DA + 1)), (0, 0))).T
    T_da = _tc_transpose128(da_padT, chunk=128)

    T_ut4 = T_ut.reshape(-1, 32)
    T_u24 = T_u2.reshape(-1, 32)
    T_v4 = T_v.reshape(-1, 32)
    T_da4 = T_da.reshape(-1, 32)

    negT = neg_job_id_lists.T.reshape(-1)
    ut, g0, g1, i_rows, da_rows, j_rows = _sc_gather(
        T_ut4, T_u24, T_v4, T_da4, users, pos_job_ids, negT, das,
        B, NNEG, DA)

    Bp = B // 4
    ut_p = ut.reshape(Bp, 128)
    g0_p = g0.reshape(Bp, 128)
    g1_p = g1.reshape(Bp, 128)
    i_p = i_rows.reshape(Bp, 128)
    da_p = da_rows.reshape(Bp, 128)
    j_p = j_rows.reshape(NNEG, Bp, 128)

    W128 = jnp.kron(jnp.eye(4, dtype=jnp.float32), W.T)
    b128 = jnp.tile(b, 4).reshape(1, 128)
    return _tc_loss(ut_p, g0_p, g1_p, i_p, da_p, j_p, W128, b128, B, NNEG)
